# baseline (speedup 1.0000x reference)
import jax
import jax.numpy as jnp
from jax.experimental import pallas as pl
from jax.experimental.pallas import tpu as pltpu

_H = 128
_K = 32
_BLOCK = 200


def _cell_kernel(nh_ref, nc_ref, fin_ref, iou_ref, uf_ref, h_ref, c_ref):
    b = nh_ref.shape[0]
    nh = nh_ref[...].reshape(b * _K, _H)
    fg = jax.lax.dot_general(
        nh, uf_ref[...], (((1,), (1,)), ((), ())),
        preferred_element_type=jnp.float32,
    )
    f = jax.nn.sigmoid(fg.reshape(b, _K, _H) + fin_ref[...][:, None, :])
    c_aggr = jnp.sum(f * nc_ref[...], axis=1)
    iou = iou_ref[...]
    i = jax.nn.sigmoid(iou[:, :_H])
    o = jax.nn.sigmoid(iou[:, _H:2 * _H])
    u = jnp.tanh(iou[:, 2 * _H:])
    c = i * u + c_aggr
    h_ref[...] = o * jnp.tanh(c)
    c_ref[...] = c


def kernel(neighbour_h, neighbour_c, f_input, iou_input, U_f):
    n, k, h = neighbour_h.shape
    b = _BLOCK
    return pl.pallas_call(
        _cell_kernel,
        grid=(n // b,),
        in_specs=[
            pl.BlockSpec((b, k, h), lambda i: (i, 0, 0)),
            pl.BlockSpec((b, k, h), lambda i: (i, 0, 0)),
            pl.BlockSpec((b, h), lambda i: (i, 0)),
            pl.BlockSpec((b, 3 * h), lambda i: (i, 0)),
            pl.BlockSpec((h, h), lambda i: (0, 0)),
        ],
        out_specs=(
            pl.BlockSpec((b, h), lambda i: (i, 0)),
            pl.BlockSpec((b, h), lambda i: (i, 0)),
        ),
        out_shape=(
            jax.ShapeDtypeStruct((n, h), jnp.float32),
            jax.ShapeDtypeStruct((n, h), jnp.float32),
        ),
        compiler_params=pltpu.CompilerParams(
            dimension_semantics=("parallel",),
        ),
    )(neighbour_h, neighbour_c, f_input, iou_input, U_f)


# speedup vs baseline: 1.0354x; 1.0354x over previous
import jax
import jax.numpy as jnp
from jax.experimental import pallas as pl
from jax.experimental.pallas import tpu as pltpu

_H = 128
_K = 32
_BLOCK = 400


def _cell_kernel(nh_ref, nc_ref, fin_ref, iou_ref, uf_ref, h_ref, c_ref):
    b = nh_ref.shape[0]
    nh = nh_ref[...].reshape(b * _K, _H)
    fg = jax.lax.dot_general(
        nh, uf_ref[...], (((1,), (1,)), ((), ())),
        preferred_element_type=jnp.float32,
    )
    f = jax.nn.sigmoid(fg.reshape(b, _K, _H) + fin_ref[...][:, None, :])
    c_aggr = jnp.sum(f * nc_ref[...], axis=1)
    iou = iou_ref[...]
    i = jax.nn.sigmoid(iou[:, :_H])
    o = jax.nn.sigmoid(iou[:, _H:2 * _H])
    u = jnp.tanh(iou[:, 2 * _H:])
    c = i * u + c_aggr
    h_ref[...] = o * jnp.tanh(c)
    c_ref[...] = c


def kernel(neighbour_h, neighbour_c, f_input, iou_input, U_f):
    n, k, h = neighbour_h.shape
    b = _BLOCK
    return pl.pallas_call(
        _cell_kernel,
        grid=(n // b,),
        in_specs=[
            pl.BlockSpec((b, k, h), lambda i: (i, 0, 0)),
            pl.BlockSpec((b, k, h), lambda i: (i, 0, 0)),
            pl.BlockSpec((b, h), lambda i: (i, 0)),
            pl.BlockSpec((b, 3 * h), lambda i: (i, 0)),
            pl.BlockSpec((h, h), lambda i: (0, 0)),
        ],
        out_specs=(
            pl.BlockSpec((b, h), lambda i: (i, 0)),
            pl.BlockSpec((b, h), lambda i: (i, 0)),
        ),
        out_shape=(
            jax.ShapeDtypeStruct((n, h), jnp.float32),
            jax.ShapeDtypeStruct((n, h), jnp.float32),
        ),
        compiler_params=pltpu.CompilerParams(
            dimension_semantics=("parallel",),
        ),
    )(neighbour_h, neighbour_c, f_input, iou_input, U_f)
